# baseline (speedup 1.0000x reference)
import jax
import jax.numpy as jnp
from jax import lax
from jax.experimental import pallas as pl
from jax.experimental.pallas import tpu as pltpu
from jax.experimental.pallas import tpu_sc as plsc

NC = 2
NS = 16
NW = NC * NS
L = 16

_B = 8192
_T = 16
_ML = 543
_BPW = _B // NW
_NGRP = _BPW // L
_STARTS = tuple(range(0, _ML - L + 1, L)) + ((_ML - L),)
_CH = 4
_PANEL = 128
_HALF = 64


def _chunks(seq, n):
    return [seq[i:i + n] for i in range(0, len(seq), n)]


def _sc_body(inp_hbm, tid_hbm, off_hbm, disc_hbm, mask_hbm, out_hbm,
             tid_v, inp_v, off_v, disc_v, mask_v, row_v, pan_v, sem):
    wid = lax.axis_index("s") * NC + lax.axis_index("c")
    base = wid * _BPW

    cps = [
        pltpu.async_copy(tid_hbm.at[pl.ds(base, _BPW)], tid_v, sem),
        pltpu.async_copy(inp_hbm.at[pl.ds(base, _BPW)], inp_v, sem),
        pltpu.async_copy(off_hbm, off_v, sem),
        pltpu.async_copy(disc_hbm, disc_v, sem),
        pltpu.async_copy(mask_hbm, mask_v, sem),
    ]
    for cp in cps:
        cp.wait()

    def fuse_row(t, _):
        for blks in _chunks(_STARTS, _CH):
            ds_ = [disc_v[t, pl.ds(st, L)] for st in blks]
            ms = [mask_v[t, pl.ds(st, L)] for st in blks]
            os_ = [off_v[t, pl.ds(st, L)] for st in blks]
            as_ = [d * m for d, m in zip(ds_, ms)]
            cs = [a * o for a, o in zip(as_, os_)]
            for st, a in zip(blks, as_):
                disc_v[t, pl.ds(st, L)] = a
            for st, c in zip(blks, cs):
                off_v[t, pl.ds(st, L)] = c
        return 0
    lax.fori_loop(0, _T, fuse_row, 0)

    iota543 = lax.iota(jnp.int32, L) * _ML

    def grp_body(g, _):
        tid16 = tid_v[pl.ds(g * L, L)]
        inp16 = inp_v[pl.ds(g * L, L)]
        for k in range(L):
            t = tid16[k]
            s = inp16[k]
            for blks in _chunks(_STARTS, _CH):
                as_ = [disc_v[t, pl.ds(st, L)] for st in blks]
                cs = [off_v[t, pl.ds(st, L)] for st in blks]
                outs = [a * s + c for a, c in zip(as_, cs)]
                for st, o in zip(blks, outs):
                    row_v[pl.ds(k * _ML + st, L)] = o

        @pl.when(g == _PANEL // L)
        def _drain():
            pltpu.make_async_copy(
                out_hbm.at[:, pl.ds(base, _PANEL)], pan_v, sem).wait()

        cb = lax.rem(g, _PANEL // L) * L
        for jc in _chunks(tuple(range(_ML)), _CH):
            vecs = [plsc.load_gather(row_v, [iota543 + j]) for j in jc]
            for j, v in zip(jc, vecs):
                pan_v[j, pl.ds(cb, L)] = v

        @pl.when(lax.rem(g, _PANEL // L) == _PANEL // L - 1)
        def _flush():
            dst = base + lax.div(g, _PANEL // L) * _PANEL
            pltpu.async_copy(pan_v, out_hbm.at[:, pl.ds(dst, _PANEL)], sem)
        return 0
    lax.fori_loop(0, _NGRP, grp_body, 0)

    pltpu.make_async_copy(
        out_hbm.at[:, pl.ds(base, _PANEL)], pan_v, sem).wait()


@jax.jit
def _sc_affine(inp1, task_ids, offsets, discrimination, mask):
    kfn = pl.kernel(
        _sc_body,
        out_type=jax.ShapeDtypeStruct((_ML, _B), jnp.float32),
        mesh=plsc.VectorSubcoreMesh(core_axis_name="c", subcore_axis_name="s"),
        compiler_params=pltpu.CompilerParams(needs_layout_passes=False),
        scratch_types=[
            pltpu.VMEM((_BPW,), jnp.int32),
            pltpu.VMEM((_BPW,), jnp.float32),
            pltpu.VMEM((_T, _ML), jnp.float32),
            pltpu.VMEM((_T, _ML), jnp.float32),
            pltpu.VMEM((_T, _ML), jnp.float32),
            pltpu.VMEM((L * _ML,), jnp.float32),
            pltpu.VMEM((_ML, _PANEL), jnp.float32),
            pltpu.SemaphoreType.DMA,
        ],
    )
    return kfn(inp1, task_ids, offsets, discrimination, mask)


def kernel(inp, task_ids, offsets, discrimination, mask):
    return _sc_affine(inp.reshape(-1), task_ids, offsets,
                      discrimination, mask).T


# speedup vs baseline: 1.8537x; 1.8537x over previous
import jax
import jax.numpy as jnp
from jax import lax
from jax.experimental import pallas as pl
from jax.experimental.pallas import tpu as pltpu
from jax.experimental.pallas import tpu_sc as plsc

NC = 2
NS = 16
NW = NC * NS
L = 16

_B = 8192
_T = 16
_ML = 543
_BPW = _B // NW
_WIN = 64
_NWIN = _BPW // _WIN
_STARTS = tuple(range(0, _ML - L + 1, L)) + ((_ML - L),)
_HALVES = (_STARTS[:17], _STARTS[17:])
_CH = 4


def _chunks(seq, n):
    return [seq[i:i + n] for i in range(0, len(seq), n)]


def _sc_body(inp_hbm, tid_hbm, off_hbm, disc_hbm, mask_hbm, out_hbm,
             tid_v, inp_v, off_v, disc_v, mask_v, out_v, ctrs, slots, sem):
    wid = lax.axis_index("s") * NC + lax.axis_index("c")
    base = wid * _BPW

    cps = [
        pltpu.async_copy(tid_hbm.at[pl.ds(base, _BPW)], tid_v, sem),
        pltpu.async_copy(inp_hbm.at[pl.ds(base, _BPW)],
                         inp_v.at[pl.ds(0, _BPW)], sem),
        pltpu.async_copy(off_hbm, off_v, sem),
        pltpu.async_copy(disc_hbm, disc_v, sem),
        pltpu.async_copy(mask_hbm, mask_v, sem),
    ]
    for cp in cps:
        cp.wait()

    def fuse_row(t, _):
        for blks in _chunks(_STARTS, _CH):
            ds_ = [disc_v[t, pl.ds(st, L)] for st in blks]
            ms = [mask_v[t, pl.ds(st, L)] for st in blks]
            os_ = [off_v[t, pl.ds(st, L)] for st in blks]
            as_ = [d * m for d, m in zip(ds_, ms)]
            cs = [a * o for a, o in zip(as_, os_)]
            for st, a in zip(blks, as_):
                disc_v[t, pl.ds(st, L)] = a
            for st, c in zip(blks, cs):
                off_v[t, pl.ds(st, L)] = c
        return 0
    lax.fori_loop(0, _T, fuse_row, 0)

    for w in range(_NWIN):
        wbase = w * _WIN
        b = w % 2

        if w >= 2:
            pltpu.make_async_copy(
                out_hbm.at[pl.ds(base, _WIN)], out_v.at[0], sem).wait()

        for t in range(_T):
            ctrs[t] = 0
        for gg in range(_WIN // L):
            tid16 = tid_v[pl.ds(wbase + gg * L, L)]
            for k in range(L):
                t = tid16[k]
                cnt = ctrs[t]
                slots[t * _WIN + cnt] = gg * L + k
                ctrs[t] = cnt + 1

        def task_body(t, _):
            cnt = ctrs[t]
            for half in _HALVES:
                areg = [disc_v[t, pl.ds(st, L)] for st in half]
                creg = [off_v[t, pl.ds(st, L)] for st in half]

                def row_body(j, _):
                    slot = slots[t * _WIN + j]
                    sv = inp_v[pl.ds(wbase + slot, L)]
                    s = sv[0]
                    for qs in _chunks(tuple(range(17)), _CH):
                        outs = [areg[q] * s + creg[q] for q in qs]
                        for q, o in zip(qs, outs):
                            out_v[b, slot, pl.ds(half[q], L)] = o
                    return 0
                lax.fori_loop(0, cnt, row_body, 0)
            return 0
        lax.fori_loop(0, _T, task_body, 0)

        pltpu.async_copy(out_v.at[b], out_hbm.at[pl.ds(base + wbase, _WIN)],
                         sem)

    for _ in range(2):
        pltpu.make_async_copy(
            out_hbm.at[pl.ds(base, _WIN)], out_v.at[0], sem).wait()


@jax.jit
def _sc_affine(inp1, task_ids, offsets, discrimination, mask):
    kfn = pl.kernel(
        _sc_body,
        out_type=jax.ShapeDtypeStruct((_B, _ML), jnp.float32),
        mesh=plsc.VectorSubcoreMesh(core_axis_name="c", subcore_axis_name="s"),
        compiler_params=pltpu.CompilerParams(needs_layout_passes=False),
        scratch_types=[
            pltpu.VMEM((_BPW,), jnp.int32),
            pltpu.VMEM((_BPW + L,), jnp.float32),
            pltpu.VMEM((_T, _ML), jnp.float32),
            pltpu.VMEM((_T, _ML), jnp.float32),
            pltpu.VMEM((_T, _ML), jnp.float32),
            pltpu.VMEM((2, _WIN, _ML), jnp.float32),
            pltpu.SMEM((_T,), jnp.int32),
            pltpu.SMEM((_T * _WIN,), jnp.int32),
            pltpu.SemaphoreType.DMA,
        ],
    )
    return kfn(inp1, task_ids, offsets, discrimination, mask)


def kernel(inp, task_ids, offsets, discrimination, mask):
    return _sc_affine(inp.reshape(-1), task_ids, offsets, discrimination,
                      mask)
